# baseline (speedup 1.0000x reference)
import functools

import jax
import jax.numpy as jnp
from jax import lax
from jax.experimental import pallas as pl
from jax.experimental.pallas import tpu as pltpu
from jax.experimental.pallas import tpu_sc as plsc

NC = 2
NS = 16
NW = NC * NS
LANES = 16

EPS = 1e-12


def _sc_gather_sum(word_ids, pos_ids, word_emb, pos_emb, chunk, nslots=2):
  n = word_ids.shape[0]
  h = word_emb.shape[1]
  b_per_w = n // NW
  nchunks = b_per_w // chunk
  mesh = plsc.VectorSubcoreMesh(core_axis_name="c", subcore_axis_name="s")

  @functools.partial(
      pl.kernel,
      mesh=mesh,
      out_type=jax.ShapeDtypeStruct((n, h), jnp.float32),
      scratch_types=(
          [pltpu.VMEM((b_per_w,), jnp.int32)] * 2
          + [pltpu.VMEM((chunk, h), jnp.float32)] * (3 * nslots)
          + [pltpu.SemaphoreType.DMA] * (3 * nslots)
      ),
  )
  def k(wids_hbm, pids_hbm, word_hbm, pos_hbm, out_hbm, *scratch):
    widx_v, pidx_v = scratch[0], scratch[1]
    bufs = scratch[2:2 + 3 * nslots]
    sems = scratch[2 + 3 * nslots:]
    wrows, prows, orows = (bufs[0:nslots], bufs[nslots:2 * nslots],
                           bufs[2 * nslots:3 * nslots])
    wsems, psems, osems = (sems[0:nslots], sems[nslots:2 * nslots],
                           sems[2 * nslots:3 * nslots])
    wid = lax.axis_index("s") * NC + lax.axis_index("c")
    base = wid * b_per_w
    pltpu.sync_copy(wids_hbm.at[pl.ds(base, b_per_w)], widx_v)
    pltpu.sync_copy(pids_hbm.at[pl.ds(base, b_per_w)], pidx_v)

    def fire_gathers(g, s):
      pltpu.async_copy(
          word_hbm.at[widx_v.at[pl.ds(g * chunk, chunk)]], wrows[s], wsems[s])
      pltpu.async_copy(
          pos_hbm.at[pidx_v.at[pl.ds(g * chunk, chunk)]], prows[s], psems[s])

    for s in range(nslots):
      fire_gathers(s, s)

    @pl.loop(0, nchunks, step=nslots)
    def _(c):
      for b in range(nslots):
        g = c + b
        pltpu.make_async_copy(
            word_hbm.at[widx_v.at[pl.ds(0, chunk)]], wrows[b], wsems[b]).wait()
        pltpu.make_async_copy(
            pos_hbm.at[pidx_v.at[pl.ds(0, chunk)]], prows[b], psems[b]).wait()

        @pl.when(g >= nslots)
        def _():
          pltpu.make_async_copy(
              orows[b], out_hbm.at[pl.ds(0, chunk)], osems[b]).wait()

        wv, pv, ov = wrows[b], prows[b], orows[b]

        @pl.loop(0, chunk)
        def _(r):
          for j in range(0, h, LANES):
            ov[r, pl.ds(j, LANES)] = wv[r, pl.ds(j, LANES)] + pv[r, pl.ds(j, LANES)]

        @pl.when(g + nslots < nchunks)
        def _():
          fire_gathers(g + nslots, b)

        pltpu.async_copy(
            ov, out_hbm.at[pl.ds(base + g * chunk, chunk)], osems[b])

    for b in range(nslots):
      pltpu.make_async_copy(
          orows[b], out_hbm.at[pl.ds(0, chunk)], osems[b]).wait()

  return k(word_ids, pos_ids, word_emb, pos_emb)


def _ln_body(x_ref, tid_ref, type_ref, gamma_ref, beta_ref, o_ref):
  x = x_ref[...]
  tid = tid_ref[0, 0, :]
  t = type_ref[...]
  tidf = tid.astype(jnp.float32)[:, None]
  e = x + t[0:1, :] + tidf * (t[1:2, :] - t[0:1, :])
  mu = jnp.mean(e, axis=-1, keepdims=True)
  d = e - mu
  var = jnp.mean(d * d, axis=-1, keepdims=True)
  normed = d * lax.rsqrt(var + EPS)
  o_ref[...] = normed * gamma_ref[...] + beta_ref[...]


def _tc_type_layernorm(summed, type_ids, type_emb, gamma, beta, tb):
  n, h = summed.shape
  nb = n // tb
  tids3 = type_ids.reshape(nb, 1, tb)
  gamma2 = gamma.reshape(1, h)
  beta2 = beta.reshape(1, h)
  return pl.pallas_call(
      _ln_body,
      grid=(nb,),
      in_specs=[
          pl.BlockSpec((tb, h), lambda i: (i, 0)),
          pl.BlockSpec((1, 1, tb), lambda i: (i, 0, 0)),
          pl.BlockSpec((2, h), lambda i: (0, 0)),
          pl.BlockSpec((1, h), lambda i: (0, 0)),
          pl.BlockSpec((1, h), lambda i: (0, 0)),
      ],
      out_specs=pl.BlockSpec((tb, h), lambda i: (i, 0)),
      out_shape=jax.ShapeDtypeStruct((n, h), jnp.float32),
  )(summed, tids3, type_emb, gamma2, beta2)


def kernel(input_ids, token_type_ids, position_ids, word_emb, pos_emb,
           type_emb, gamma, beta):
  b, s = input_ids.shape
  h = word_emb.shape[1]
  wids = input_ids.reshape(-1).astype(jnp.int32)
  pids = position_ids.reshape(-1).astype(jnp.int32)
  tids = token_type_ids.reshape(-1).astype(jnp.int32)
  summed = _sc_gather_sum(wids, pids, word_emb, pos_emb, chunk=8, nslots=4)
  out = _tc_type_layernorm(summed, tids, type_emb, gamma, beta, tb=2048)
  return out.reshape(b, s, h)


# speedup vs baseline: 1.0204x; 1.0204x over previous
import functools

import jax
import jax.numpy as jnp
from jax import lax
from jax.experimental import pallas as pl
from jax.experimental.pallas import tpu as pltpu
from jax.experimental.pallas import tpu_sc as plsc

NC = 2
NS = 16
NW = NC * NS
LANES = 16

EPS = 1e-12


def _sc_gather_sum(word_ids, pos_ids, word_emb, pos_emb, chunk, nslots=2):
  n = word_ids.shape[0]
  h = word_emb.shape[1]
  b_per_w = n // NW
  nchunks = b_per_w // chunk
  mesh = plsc.VectorSubcoreMesh(core_axis_name="c", subcore_axis_name="s")

  @functools.partial(
      pl.kernel,
      mesh=mesh,
      out_type=jax.ShapeDtypeStruct((n, h), jnp.float32),
      scratch_types=(
          [pltpu.VMEM((b_per_w,), jnp.int32)] * 2
          + [pltpu.VMEM((chunk, h), jnp.float32)] * (3 * nslots)
          + [pltpu.SemaphoreType.DMA] * (3 * nslots)
      ),
  )
  def k(wids_hbm, pids_hbm, word_hbm, pos_hbm, out_hbm, *scratch):
    widx_v, pidx_v = scratch[0], scratch[1]
    bufs = scratch[2:2 + 3 * nslots]
    sems = scratch[2 + 3 * nslots:]
    wrows, prows, orows = (bufs[0:nslots], bufs[nslots:2 * nslots],
                           bufs[2 * nslots:3 * nslots])
    wsems, psems, osems = (sems[0:nslots], sems[nslots:2 * nslots],
                           sems[2 * nslots:3 * nslots])
    wid = lax.axis_index("s") * NC + lax.axis_index("c")
    base = wid * b_per_w
    pltpu.sync_copy(wids_hbm.at[pl.ds(base, b_per_w)], widx_v)
    pltpu.sync_copy(pids_hbm.at[pl.ds(base, b_per_w)], pidx_v)

    def fire_gathers(g, s):
      pltpu.async_copy(
          word_hbm.at[widx_v.at[pl.ds(g * chunk, chunk)]], wrows[s], wsems[s])
      pltpu.async_copy(
          pos_hbm.at[pidx_v.at[pl.ds(g * chunk, chunk)]], prows[s], psems[s])

    for s in range(nslots):
      fire_gathers(s, s)

    @pl.loop(0, nchunks, step=nslots)
    def _(c):
      for b in range(nslots):
        g = c + b
        pltpu.make_async_copy(
            word_hbm.at[widx_v.at[pl.ds(0, chunk)]], wrows[b], wsems[b]).wait()
        pltpu.make_async_copy(
            pos_hbm.at[pidx_v.at[pl.ds(0, chunk)]], prows[b], psems[b]).wait()

        @pl.when(g >= nslots)
        def _():
          pltpu.make_async_copy(
              orows[b], out_hbm.at[pl.ds(0, chunk)], osems[b]).wait()

        wv, pv, ov = wrows[b], prows[b], orows[b]

        @pl.loop(0, chunk)
        def _(r):
          for j in range(0, h, LANES):
            ov[r, pl.ds(j, LANES)] = wv[r, pl.ds(j, LANES)] + pv[r, pl.ds(j, LANES)]

        @pl.when(g + nslots < nchunks)
        def _():
          fire_gathers(g + nslots, b)

        pltpu.async_copy(
            ov, out_hbm.at[pl.ds(base + g * chunk, chunk)], osems[b])

    for b in range(nslots):
      pltpu.make_async_copy(
          orows[b], out_hbm.at[pl.ds(0, chunk)], osems[b]).wait()

  return k(word_ids, pos_ids, word_emb, pos_emb)


def _ln_body(x_ref, tid_ref, type_ref, gamma_ref, beta_ref, o_ref):
  x = x_ref[...]
  tid = tid_ref[0, 0, :]
  t = type_ref[...]
  tidf = tid.astype(jnp.float32)[:, None]
  e = x + t[0:1, :] + tidf * (t[1:2, :] - t[0:1, :])
  mu = jnp.mean(e, axis=-1, keepdims=True)
  d = e - mu
  var = jnp.mean(d * d, axis=-1, keepdims=True)
  normed = d * lax.rsqrt(var + EPS)
  o_ref[...] = normed * gamma_ref[...] + beta_ref[...]


def _tc_type_layernorm(summed, type_ids, type_emb, gamma, beta, tb):
  n, h = summed.shape
  nb = n // tb
  tids3 = type_ids.reshape(nb, 1, tb)
  gamma2 = gamma.reshape(1, h)
  beta2 = beta.reshape(1, h)
  return pl.pallas_call(
      _ln_body,
      grid=(nb,),
      in_specs=[
          pl.BlockSpec((tb, h), lambda i: (i, 0)),
          pl.BlockSpec((1, 1, tb), lambda i: (i, 0, 0)),
          pl.BlockSpec((2, h), lambda i: (0, 0)),
          pl.BlockSpec((1, h), lambda i: (0, 0)),
          pl.BlockSpec((1, h), lambda i: (0, 0)),
      ],
      out_specs=pl.BlockSpec((tb, h), lambda i: (i, 0)),
      out_shape=jax.ShapeDtypeStruct((n, h), jnp.float32),
  )(summed, tids3, type_emb, gamma2, beta2)


def kernel(input_ids, token_type_ids, position_ids, word_emb, pos_emb,
           type_emb, gamma, beta):
  b, s = input_ids.shape
  h = word_emb.shape[1]
  wids = input_ids.reshape(-1).astype(jnp.int32)
  pids = position_ids.reshape(-1).astype(jnp.int32)
  tids = token_type_ids.reshape(-1).astype(jnp.int32)
  summed = _sc_gather_sum(wids, pids, word_emb, pos_emb, chunk=16, nslots=2)
  out = _tc_type_layernorm(summed, tids, type_emb, gamma, beta, tb=2048)
  return out.reshape(b, s, h)
